# Initial kernel scaffold; baseline (speedup 1.0000x reference)
import functools

import jax
import jax.numpy as jnp
from jax import lax
from jax.experimental import pallas as pl
from jax.experimental.pallas import tpu as pltpu
from jax.experimental.pallas import tpu_sc as plsc

BATCH = 16384
HIST = 50
DIM = 64
NUM_IDX = BATCH * HIST
GROUP = 128
NC = 2
NS = 16
NW = NC * NS
GROUPS_PER_W = NUM_IDX // (GROUP * NW)
NBUF = 5
NCHUNK = GROUPS_PER_W // NBUF


def _body(idx_hbm, table_hbm, out_hbm, idx_v, rows_v, gsem):
    wid = lax.axis_index("s") * NC + lax.axis_index("c")
    base_g = wid * GROUPS_PER_W

    pltpu.sync_copy(idx_hbm.at[pl.ds(base_g, GROUPS_PER_W)], idx_v)

    @pl.loop(0, NCHUNK)
    def _chunk(c):
        g0 = c * NBUF
        copies = []
        for b in range(NBUF):
            copies.append(
                pltpu.async_copy(
                    table_hbm.at[idx_v.at[g0 + b]],
                    rows_v.at[pl.ds(b * GROUP, GROUP)],
                    gsem,
                )
            )
        for cp in copies:
            cp.wait()
        pltpu.sync_copy(
            rows_v,
            out_hbm.at[pl.ds((base_g + g0) * GROUP, NBUF * GROUP)],
        )


def kernel(x, table):
    idx = x.reshape(NUM_IDX // GROUP, GROUP).astype(jnp.int32)
    mesh = plsc.VectorSubcoreMesh(core_axis_name="c", subcore_axis_name="s")
    grab = pl.kernel(
        _body,
        out_type=jax.ShapeDtypeStruct((NUM_IDX, DIM), jnp.float32),
        mesh=mesh,
        scratch_types=[
            pltpu.VMEM((GROUPS_PER_W, GROUP), jnp.int32),
            pltpu.VMEM((NBUF * GROUP, DIM), jnp.float32),
            pltpu.SemaphoreType.DMA,
        ],
    )
    out = grab(idx, table)
    return out.reshape(BATCH, HIST, DIM)


# speedup vs baseline: 1.8413x; 1.8413x over previous
import functools

import jax
import jax.numpy as jnp
from jax import lax
from jax.experimental import pallas as pl
from jax.experimental.pallas import tpu as pltpu
from jax.experimental.pallas import tpu_sc as plsc

BATCH = 16384
HIST = 50
DIM = 64
NUM_IDX = BATCH * HIST
GROUP = 128
NC = 2
NS = 16
NW = NC * NS
GROUPS_PER_W = NUM_IDX // (GROUP * NW)
NBUF = 5
NCHUNK = GROUPS_PER_W // NBUF


def _body(idx_hbm, table_hbm, out_hbm, idx_v, rows_v, gsem):
    wid = lax.axis_index("s") * NC + lax.axis_index("c")
    base_g = wid * GROUPS_PER_W

    pltpu.sync_copy(idx_hbm.at[pl.ds(base_g, GROUPS_PER_W)], idx_v)

    @pl.loop(0, NCHUNK)
    def _chunk(c):
        g0 = c * NBUF
        copies = []
        for b in range(NBUF):
            copies.append(
                pltpu.async_copy(
                    table_hbm.at[idx_v.at[g0 + b]],
                    rows_v.at[pl.ds(b * GROUP, GROUP)],
                    gsem,
                )
            )
        for cp in copies:
            cp.wait()
        pltpu.sync_copy(
            rows_v,
            out_hbm.at[pl.ds((base_g + g0) * GROUP, NBUF * GROUP)],
        )


def kernel(x, table):
    idx = x.reshape(NUM_IDX // GROUP, GROUP).astype(jnp.int32)
    mesh = plsc.VectorSubcoreMesh(core_axis_name="c", subcore_axis_name="s")
    grab = pl.kernel(
        _body,
        out_type=jax.ShapeDtypeStruct((NUM_IDX, DIM), jnp.float32),
        mesh=mesh,
        scratch_types=[
            pltpu.VMEM((GROUPS_PER_W, GROUP), jnp.int32),
            pltpu.VMEM((NBUF * GROUP, DIM), jnp.float32),
            pltpu.SemaphoreType.DMA,
        ],
        compiler_params=pltpu.CompilerParams(use_tc_tiling_on_sc=False),
    )
    out = grab(idx, table)
    return out.reshape(BATCH, HIST, DIM)


# speedup vs baseline: 1.8738x; 1.0177x over previous
import functools

import jax
import jax.numpy as jnp
from jax import lax
from jax.experimental import pallas as pl
from jax.experimental.pallas import tpu as pltpu
from jax.experimental.pallas import tpu_sc as plsc

BATCH = 16384
HIST = 50
DIM = 64
NUM_IDX = BATCH * HIST
GROUP = 128
NC = 2
NS = 16
NW = NC * NS
GROUPS_PER_W = NUM_IDX // (GROUP * NW)
NBUF = 5
NCHUNK = GROUPS_PER_W // NBUF


def _body(idx_hbm, table_hbm, out_hbm, idx_v, rows_v, gsem):
    wid = lax.axis_index("s") * NC + lax.axis_index("c")
    base_g = wid * GROUPS_PER_W

    pltpu.sync_copy(idx_hbm.at[pl.ds(base_g, GROUPS_PER_W)], idx_v)

    def fire(cc, s):
        g0 = cc * NBUF
        for b in range(NBUF):
            pltpu.async_copy(
                table_hbm.at[idx_v.at[g0 + b]],
                rows_v.at[s].at[pl.ds(b * GROUP, GROUP)],
                gsem.at[s],
            )

    fire(0, 0)

    @pl.loop(0, NCHUNK, step=2)
    def _chunk(c):
        for s in range(2):
            cc = c + s

            @pl.when(cc < NCHUNK - 1)
            def _():
                fire(cc + 1, 1 - s)

            pltpu.make_async_copy(
                table_hbm.at[pl.ds(0, NBUF * GROUP)],
                rows_v.at[s],
                gsem.at[s],
            ).wait()
            pltpu.sync_copy(
                rows_v.at[s],
                out_hbm.at[pl.ds((base_g + cc * NBUF) * GROUP, NBUF * GROUP)],
            )


def kernel(x, table):
    idx = x.reshape(NUM_IDX // GROUP, GROUP).astype(jnp.int32)
    mesh = plsc.VectorSubcoreMesh(core_axis_name="c", subcore_axis_name="s")
    grab = pl.kernel(
        _body,
        out_type=jax.ShapeDtypeStruct((NUM_IDX, DIM), jnp.float32),
        mesh=mesh,
        scratch_types=[
            pltpu.VMEM((GROUPS_PER_W, GROUP), jnp.int32),
            pltpu.VMEM((2, NBUF * GROUP, DIM), jnp.float32),
            pltpu.SemaphoreType.DMA((2,)),
        ],
        compiler_params=pltpu.CompilerParams(use_tc_tiling_on_sc=False),
    )
    out = grab(idx, table)
    return out.reshape(BATCH, HIST, DIM)
